# Initial kernel scaffold; baseline (speedup 1.0000x reference)
import jax
import jax.numpy as jnp
from jax.experimental import pallas as pl


def kernel(x, edge_index, W1, b1, W2, b2, W_out, b_out):
    raise NotImplementedError("write your pallas kernel here")


# speedup vs baseline: 7.0603x; 7.0603x over previous
import functools

import jax
import jax.numpy as jnp
from jax import lax
from jax.experimental import pallas as pl
from jax.experimental.pallas import tpu as pltpu
from jax.experimental.pallas import tpu_sc as plsc

N = 10000
NPAD = 10240
STRIPE = NPAD // 16
CW = 128
B = 128
NB = 80
EPAD = 16 * NB * B
DEGW = 128
BLK = 1000

_mesh = plsc.VectorSubcoreMesh(core_axis_name="c", subcore_axis_name="s")


@functools.partial(
    pl.kernel,
    out_type=jax.ShapeDtypeStruct((2 * NPAD, DEGW), jnp.float32),
    mesh=_mesh,
    scratch_types=[
        pltpu.VMEM((NB // 2, B), jnp.int32),
        pltpu.VMEM((B, DEGW), jnp.float32),
        pltpu.VMEM((B, DEGW), jnp.float32),
        pltpu.VMEM_SHARED((NPAD, DEGW), jnp.float32),
    ],
)
def _deg_kernel(ddeg, ones_h, zero_h, out, idxv, ones_v, bncd, dacc):
    c = lax.axis_index("c")
    s = lax.axis_index("s")
    w = c * 16 + s
    pltpu.sync_copy(ddeg.at[w], idxv)
    pltpu.sync_copy(ones_h, ones_v)
    pltpu.sync_copy(zero_h, bncd)

    def z_body(i, carry):
        pltpu.sync_copy(bncd, dacc.at[pl.ds(s * STRIPE + i * B, B)])
        return carry

    lax.fori_loop(0, STRIPE // B, z_body, 0)
    plsc.subcore_barrier()

    def e_body(j, carry):
        pltpu.sync_copy(ones_v, dacc.at[idxv.at[j]], add=True)
        return carry

    lax.fori_loop(0, NB // 2, e_body, 0)
    plsc.subcore_barrier()

    def out_b(i, carry):
        pltpu.sync_copy(dacc.at[pl.ds(s * STRIPE + i * B, B)], bncd)
        pltpu.sync_copy(bncd, out.at[pl.ds(c * NPAD + s * STRIPE + i * B, B)])
        return carry

    lax.fori_loop(0, STRIPE // B, out_b, 0)


def _make_scatter(P):
    pc = P // 2

    @functools.partial(
        pl.kernel,
        out_type=jax.ShapeDtypeStruct((P * NPAD, CW), jnp.float32),
        mesh=_mesh,
        scratch_types=[
            pltpu.VMEM((NB, B), jnp.int32),
            pltpu.VMEM((NB, B), jnp.int32),
            pltpu.VMEM((NB, B), jnp.int32),
            pltpu.VMEM((B, CW), jnp.float32),
            pltpu.VMEM_SHARED((NPAD, CW), jnp.float32),
            pltpu.SemaphoreType.DMA,
        ],
    )
    def _scatter(xs, srcr, dstr, out, idxs, idxc, idxd, msg, acc, gsem):
        c = lax.axis_index("c")
        s = lax.axis_index("s")
        pltpu.sync_copy(srcr.at[s], idxs)
        pltpu.sync_copy(dstr.at[s], idxd)
        for pi in range(pc):
            p = c * pc + pi
            base = p * NPAD

            def off_r(r, carry):
                def off_k(k, carry2):
                    idxc[r, pl.ds(k * 16, 16)] = (
                        idxs[r, pl.ds(k * 16, 16)] + base)
                    return carry2
                return lax.fori_loop(0, B // 16, off_k, carry)

            lax.fori_loop(0, NB, off_r, 0)

            def init_b(i, carry):
                pltpu.sync_copy(xs.at[pl.ds(base + s * STRIPE + i * B, B)], msg)
                pltpu.sync_copy(msg, acc.at[pl.ds(s * STRIPE + i * B, B)])
                return carry

            lax.fori_loop(0, STRIPE // B, init_b, 0)
            plsc.subcore_barrier()

            def e_body(j, carry):
                pltpu.async_copy(xs.at[idxc.at[j]], msg, gsem).wait()
                pltpu.sync_copy(msg, acc.at[idxd.at[j]], add=True)
                return carry

            lax.fori_loop(0, NB, e_body, 0)
            plsc.subcore_barrier()

            def out_b(i, carry):
                pltpu.sync_copy(acc.at[pl.ds(s * STRIPE + i * B, B)], msg)
                pltpu.sync_copy(msg, out.at[pl.ds(base + s * STRIPE + i * B, B)])
                return carry

            lax.fori_loop(0, STRIPE // B, out_b, 0)

    return _scatter


_scatter2 = _make_scatter(2)
_scatter4 = _make_scatter(4)


def _dis_of(degp_ref):
    deg = jnp.sum(degp_ref[0] + degp_ref[1], axis=1) * (1.0 / DEGW) + 1.0
    return lax.rsqrt(deg)


def _k0_body(degp_ref, x_ref, xs1_ref):
    dis = _dis_of(degp_ref)
    xs = x_ref[...] * dis[:, None]
    for k in range(2):
        xs1_ref[k] = xs[:, k * CW:(k + 1) * CW]


def _ka_body(degp_ref, acc1_ref, w1_ref, b1_ref, xs2_ref):
    dis = _dis_of(degp_ref)
    s1 = jnp.concatenate([acc1_ref[k] for k in range(2)], axis=1) * dis[:, None]
    h = jnp.dot(s1, w1_ref[...], preferred_element_type=jnp.float32) + b1_ref[...]
    h = jnp.maximum(h, 0.0)
    xs2 = h * dis[:, None]
    for k in range(4):
        xs2_ref[k] = xs2[:, k * CW:(k + 1) * CW]


def _kb_body(degp_ref, acc2_ref, w2_ref, b2_ref, wr_ref, bo_ref, out_ref):
    dis = _dis_of(degp_ref)
    s2 = jnp.concatenate([acc2_ref[k] for k in range(4)], axis=1) * dis[:, None]
    h = jnp.dot(s2, w2_ref[...], preferred_element_type=jnp.float32) + b2_ref[...]
    h = jnp.maximum(h, 0.0)
    hg = h.reshape(BLK // 100, 100, 512)
    vals = jnp.sum(hg * wr_ref[...][None], axis=(1, 2))
    out_ref[0] = jnp.broadcast_to(vals[:, None], (BLK // 100, 128)) + bo_ref[0, 0]


_degp_spec = pl.BlockSpec((2, BLK, DEGW), lambda i: (0, i, 0))


def _k0(degp, x):
    return pl.pallas_call(
        _k0_body,
        grid=(N // BLK,),
        in_specs=[_degp_spec, pl.BlockSpec((BLK, 256), lambda i: (i, 0))],
        out_specs=pl.BlockSpec((2, BLK, CW), lambda i: (0, i, 0)),
        out_shape=jax.ShapeDtypeStruct((2, NPAD, CW), jnp.float32),
    )(degp, x)


def _ka(degp, acc1, W1, b1):
    return pl.pallas_call(
        _ka_body,
        grid=(N // BLK,),
        in_specs=[
            _degp_spec,
            pl.BlockSpec((2, BLK, CW), lambda i: (0, i, 0)),
            pl.BlockSpec((256, 512), lambda i: (0, 0)),
            pl.BlockSpec((1, 512), lambda i: (0, 0)),
        ],
        out_specs=pl.BlockSpec((4, BLK, CW), lambda i: (0, i, 0)),
        out_shape=jax.ShapeDtypeStruct((4, NPAD, CW), jnp.float32),
    )(degp, acc1, W1, b1)


def _kb(degp, acc2, W2, b2, wr, bo):
    return pl.pallas_call(
        _kb_body,
        grid=(N // BLK,),
        in_specs=[
            _degp_spec,
            pl.BlockSpec((4, BLK, CW), lambda i: (0, i, 0)),
            pl.BlockSpec((512, 512), lambda i: (0, 0)),
            pl.BlockSpec((1, 512), lambda i: (0, 0)),
            pl.BlockSpec((100, 512), lambda i: (0, 0)),
            pl.BlockSpec((1, 1), lambda i: (0, 0)),
        ],
        out_specs=pl.BlockSpec((1, BLK // 100, 128), lambda i: (i, 0, 0)),
        out_shape=jax.ShapeDtypeStruct((N // BLK, BLK // 100, 128), jnp.float32),
    )(degp, acc2, W2, b2, wr, bo)


def kernel(x, edge_index, W1, b1, W2, b2, W_out, b_out):
    src = edge_index[0].astype(jnp.int32)
    dst = edge_index[1].astype(jnp.int32)
    npad = EPAD - src.shape[0]
    src_p = jnp.concatenate([src, jnp.zeros((npad,), jnp.int32)])
    dst_p = jnp.concatenate([dst, jnp.full((npad,), N, jnp.int32)])
    srcr = src_p.reshape(16, NB, B)
    dstr = dst_p.reshape(16, NB, B)
    ddeg = dst_p.reshape(32, NB // 2, B)

    ones_h = jnp.ones((B, DEGW), jnp.float32)
    zero_h = jnp.zeros((B, DEGW), jnp.float32)
    degp = _deg_kernel(ddeg, ones_h, zero_h).reshape(2, NPAD, DEGW)
    xs1 = _k0(degp, x).reshape(2 * NPAD, CW)
    acc1 = _scatter2(xs1, srcr, dstr).reshape(2, NPAD, CW)
    xs2 = _ka(degp, acc1, W1, b1.reshape(1, 512)).reshape(4 * NPAD, CW)
    acc2 = _scatter4(xs2, srcr, dstr).reshape(4, NPAD, CW)
    res = _kb(degp, acc2, W2, b2.reshape(1, 512),
              W_out.reshape(100, 512), b_out.reshape(1, 1))
    return res.reshape(100, 128)[:, :1]


# speedup vs baseline: 7.0635x; 1.0005x over previous
import functools

import jax
import jax.numpy as jnp
from jax import lax
from jax.experimental import pallas as pl
from jax.experimental.pallas import tpu as pltpu
from jax.experimental.pallas import tpu_sc as plsc

N = 10000
NPAD = 10240
STRIPE = NPAD // 16
CW = 128
B = 128
NB = 80
EPAD = 16 * NB * B
DEGW = 128
BLK = 1000

_mesh = plsc.VectorSubcoreMesh(core_axis_name="c", subcore_axis_name="s")


@functools.partial(
    pl.kernel,
    out_type=jax.ShapeDtypeStruct((2 * NPAD, DEGW), jnp.float32),
    mesh=_mesh,
    scratch_types=[
        pltpu.VMEM((NB // 2, B), jnp.int32),
        pltpu.VMEM((B, DEGW), jnp.float32),
        pltpu.VMEM((B, DEGW), jnp.float32),
        pltpu.VMEM_SHARED((NPAD, DEGW), jnp.float32),
    ],
)
def _deg_kernel(ddeg, ones_h, zero_h, out, idxv, ones_v, bncd, dacc):
    c = lax.axis_index("c")
    s = lax.axis_index("s")
    w = c * 16 + s
    pltpu.sync_copy(ddeg.at[w], idxv)
    pltpu.sync_copy(ones_h, ones_v)
    pltpu.sync_copy(zero_h, bncd)

    def z_body(i, carry):
        pltpu.sync_copy(bncd, dacc.at[pl.ds(s * STRIPE + i * B, B)])
        return carry

    lax.fori_loop(0, STRIPE // B, z_body, 0)
    plsc.subcore_barrier()

    def e_body(j, carry):
        pltpu.sync_copy(ones_v, dacc.at[idxv.at[j]], add=True)
        return carry

    lax.fori_loop(0, NB // 2, e_body, 0)
    plsc.subcore_barrier()

    def out_b(i, carry):
        pltpu.sync_copy(dacc.at[pl.ds(s * STRIPE + i * B, B)], bncd)
        pltpu.sync_copy(bncd, out.at[pl.ds(c * NPAD + s * STRIPE + i * B, B)])
        return carry

    lax.fori_loop(0, STRIPE // B, out_b, 0)


def _make_scatter(P):
    pc = P // 2

    @functools.partial(
        pl.kernel,
        out_type=jax.ShapeDtypeStruct((P * NPAD, CW), jnp.float32),
        mesh=_mesh,
        scratch_types=[
            pltpu.VMEM((NB, B), jnp.int32),
            pltpu.VMEM((NB, B), jnp.int32),
            pltpu.VMEM((NB, B), jnp.int32),
            pltpu.VMEM((B, CW), jnp.float32),
            pltpu.VMEM_SHARED((NPAD, CW), jnp.float32),
            pltpu.SemaphoreType.DMA,
        ],
    )
    def _scatter(xs, srcr, dstr, out, idxs, idxc, idxd, msg, acc, gsem):
        c = lax.axis_index("c")
        s = lax.axis_index("s")
        pltpu.sync_copy(srcr.at[s], idxs)
        pltpu.sync_copy(dstr.at[s], idxd)
        for pi in range(pc):
            p = c * pc + pi
            base = p * NPAD

            def off_r(r, carry):
                def off_k(k, carry2):
                    idxc[r, pl.ds(k * 16, 16)] = (
                        idxs[r, pl.ds(k * 16, 16)] + base)
                    return carry2
                return lax.fori_loop(0, B // 16, off_k, carry)

            lax.fori_loop(0, NB, off_r, 0)

            def init_b(i, carry):
                pltpu.sync_copy(xs.at[pl.ds(base + s * STRIPE + i * B, B)], msg)
                pltpu.sync_copy(msg, acc.at[pl.ds(s * STRIPE + i * B, B)])
                return carry

            lax.fori_loop(0, STRIPE // B, init_b, 0)
            plsc.subcore_barrier()

            def e_body(j, carry):
                pltpu.async_copy(xs.at[idxc.at[j]], msg, gsem).wait()
                pltpu.sync_copy(msg, acc.at[idxd.at[j]], add=True)
                return carry

            lax.fori_loop(0, NB, e_body, 0)
            plsc.subcore_barrier()

            def out_b(i, carry):
                pltpu.sync_copy(acc.at[pl.ds(s * STRIPE + i * B, B)], msg)
                pltpu.sync_copy(msg, out.at[pl.ds(base + s * STRIPE + i * B, B)])
                return carry

            lax.fori_loop(0, STRIPE // B, out_b, 0)

    return _scatter


_scatter2 = _make_scatter(2)
_scatter4 = _make_scatter(4)


def _dis_of(degp_ref):
    deg = jnp.sum(degp_ref[0] + degp_ref[1], axis=1) * (1.0 / DEGW) + 1.0
    return 1.0 / jnp.sqrt(deg)


def _k0_body(degp_ref, x_ref, xs1_ref):
    dis = _dis_of(degp_ref)
    xs = x_ref[...] * dis[:, None]
    for k in range(2):
        xs1_ref[k] = xs[:, k * CW:(k + 1) * CW]


def _ka_body(degp_ref, acc1_ref, w1_ref, b1_ref, xs2_ref):
    dis = _dis_of(degp_ref)
    s1 = jnp.concatenate([acc1_ref[k] for k in range(2)], axis=1) * dis[:, None]
    h = jnp.dot(s1, w1_ref[...], preferred_element_type=jnp.float32) + b1_ref[...]
    h = jnp.maximum(h, 0.0)
    xs2 = h * dis[:, None]
    for k in range(4):
        xs2_ref[k] = xs2[:, k * CW:(k + 1) * CW]


def _kb_body(degp_ref, acc2_ref, w2_ref, b2_ref, wr_ref, bo_ref, out_ref):
    dis = _dis_of(degp_ref)
    s2 = jnp.concatenate([acc2_ref[k] for k in range(4)], axis=1) * dis[:, None]
    h = jnp.dot(s2, w2_ref[...], preferred_element_type=jnp.float32) + b2_ref[...]
    h = jnp.maximum(h, 0.0)
    hg = h.reshape(BLK // 100, 100, 512)
    vals = jnp.sum(hg * wr_ref[...][None], axis=(1, 2))
    out_ref[0] = jnp.broadcast_to(vals[:, None], (BLK // 100, 128)) + bo_ref[0, 0]


_degp_spec = pl.BlockSpec((2, BLK, DEGW), lambda i: (0, i, 0))


def _k0(degp, x):
    return pl.pallas_call(
        _k0_body,
        grid=(N // BLK,),
        in_specs=[_degp_spec, pl.BlockSpec((BLK, 256), lambda i: (i, 0))],
        out_specs=pl.BlockSpec((2, BLK, CW), lambda i: (0, i, 0)),
        out_shape=jax.ShapeDtypeStruct((2, NPAD, CW), jnp.float32),
    )(degp, x)


def _ka(degp, acc1, W1, b1):
    return pl.pallas_call(
        _ka_body,
        grid=(N // BLK,),
        in_specs=[
            _degp_spec,
            pl.BlockSpec((2, BLK, CW), lambda i: (0, i, 0)),
            pl.BlockSpec((256, 512), lambda i: (0, 0)),
            pl.BlockSpec((1, 512), lambda i: (0, 0)),
        ],
        out_specs=pl.BlockSpec((4, BLK, CW), lambda i: (0, i, 0)),
        out_shape=jax.ShapeDtypeStruct((4, NPAD, CW), jnp.float32),
    )(degp, acc1, W1, b1)


def _kb(degp, acc2, W2, b2, wr, bo):
    return pl.pallas_call(
        _kb_body,
        grid=(N // BLK,),
        in_specs=[
            _degp_spec,
            pl.BlockSpec((4, BLK, CW), lambda i: (0, i, 0)),
            pl.BlockSpec((512, 512), lambda i: (0, 0)),
            pl.BlockSpec((1, 512), lambda i: (0, 0)),
            pl.BlockSpec((100, 512), lambda i: (0, 0)),
            pl.BlockSpec((1, 1), lambda i: (0, 0)),
        ],
        out_specs=pl.BlockSpec((1, BLK // 100, 128), lambda i: (i, 0, 0)),
        out_shape=jax.ShapeDtypeStruct((N // BLK, BLK // 100, 128), jnp.float32),
    )(degp, acc2, W2, b2, wr, bo)


def kernel(x, edge_index, W1, b1, W2, b2, W_out, b_out):
    src = edge_index[0].astype(jnp.int32)
    dst = edge_index[1].astype(jnp.int32)
    npad = EPAD - src.shape[0]
    src_p = jnp.concatenate([src, jnp.zeros((npad,), jnp.int32)])
    dst_p = jnp.concatenate([dst, jnp.full((npad,), N, jnp.int32)])
    srcr = src_p.reshape(16, NB, B)
    dstr = dst_p.reshape(16, NB, B)
    ddeg = dst_p.reshape(32, NB // 2, B)

    ones_h = jnp.ones((B, DEGW), jnp.float32)
    zero_h = jnp.zeros((B, DEGW), jnp.float32)
    degp = _deg_kernel(ddeg, ones_h, zero_h).reshape(2, NPAD, DEGW)
    xs1 = _k0(degp, x).reshape(2 * NPAD, CW)
    acc1 = _scatter2(xs1, srcr, dstr).reshape(2, NPAD, CW)
    xs2 = _ka(degp, acc1, W1, b1.reshape(1, 512)).reshape(4 * NPAD, CW)
    acc2 = _scatter4(xs2, srcr, dstr).reshape(4, NPAD, CW)
    res = _kb(degp, acc2, W2, b2.reshape(1, 512),
              W_out.reshape(100, 512), b_out.reshape(1, 1))
    return res.reshape(100, 128)[:, :1]


# speedup vs baseline: 7.8516x; 1.1116x over previous
import functools

import jax
import jax.numpy as jnp
from jax import lax
from jax.experimental import pallas as pl
from jax.experimental.pallas import tpu as pltpu
from jax.experimental.pallas import tpu_sc as plsc

N = 10000
NPAD = 10240
STRIPE = NPAD // 16
CW = 128
B = 128
NB = 80
EPAD = 16 * NB * B
DEGW = 128
BLK = 1000

_mesh = plsc.VectorSubcoreMesh(core_axis_name="c", subcore_axis_name="s")


@functools.partial(
    pl.kernel,
    out_type=jax.ShapeDtypeStruct((2 * NPAD, DEGW), jnp.float32),
    mesh=_mesh,
    scratch_types=[
        pltpu.VMEM((NB // 2, B), jnp.int32),
        pltpu.VMEM((B, DEGW), jnp.float32),
        pltpu.VMEM((B, DEGW), jnp.float32),
        pltpu.VMEM_SHARED((NPAD, DEGW), jnp.float32),
    ],
)
def _deg_kernel(ddeg, ones_h, zero_h, out, idxv, ones_v, bncd, dacc):
    c = lax.axis_index("c")
    s = lax.axis_index("s")
    w = c * 16 + s
    pltpu.sync_copy(ddeg.at[w], idxv)
    pltpu.sync_copy(ones_h, ones_v)
    pltpu.sync_copy(zero_h, bncd)

    def z_body(i, carry):
        pltpu.sync_copy(bncd, dacc.at[pl.ds(s * STRIPE + i * B, B)])
        return carry

    lax.fori_loop(0, STRIPE // B, z_body, 0)
    plsc.subcore_barrier()

    def e_body(j, carry):
        pltpu.sync_copy(ones_v, dacc.at[idxv.at[j]], add=True)
        return carry

    lax.fori_loop(0, NB // 2, e_body, 0)
    plsc.subcore_barrier()

    def out_b(i, carry):
        pltpu.sync_copy(dacc.at[pl.ds(s * STRIPE + i * B, B)], bncd)
        pltpu.sync_copy(bncd, out.at[pl.ds(c * NPAD + s * STRIPE + i * B, B)])
        return carry

    lax.fori_loop(0, STRIPE // B, out_b, 0)


def _make_scatter(P):
    pc = P // 2
    W = 40

    @functools.partial(
        pl.kernel,
        out_type=jax.ShapeDtypeStruct((P * NPAD, CW), jnp.float32),
        mesh=_mesh,
        scratch_types=[
            pltpu.VMEM((W, B), jnp.int32),
            pltpu.VMEM((W, B), jnp.int32),
            pltpu.VMEM((2, B, CW), jnp.float32),
            pltpu.VMEM_SHARED((NPAD, CW), jnp.float32),
            pltpu.SemaphoreType.DMA,
            pltpu.SemaphoreType.DMA,
        ],
    )
    def _scatter(xs, srcr, dstr, out, idxg, idxd, msg, acc, g0, g1):
        c = lax.axis_index("c")
        s = lax.axis_index("s")
        for pi in range(pc):
            p = c * pc + pi
            base = p * NPAD

            def init_b(i, carry):
                pltpu.sync_copy(xs.at[pl.ds(base + s * STRIPE + i * B, B)],
                                msg.at[0])
                pltpu.sync_copy(msg.at[0], acc.at[pl.ds(s * STRIPE + i * B, B)])
                return carry

            lax.fori_loop(0, STRIPE // B, init_b, 0)
            plsc.subcore_barrier()

            for wi in range(NB // W):
                pltpu.sync_copy(srcr.at[s].at[pl.ds(wi * W, W)], idxg)
                pltpu.sync_copy(dstr.at[s].at[pl.ds(wi * W, W)], idxd)

                def off_r(r, carry):
                    def off_k(k, carry2):
                        idxg[r, pl.ds(k * 16, 16)] = (
                            idxg[r, pl.ds(k * 16, 16)] + base)
                        return carry2
                    return lax.fori_loop(0, B // 16, off_k, carry)

                lax.fori_loop(0, W, off_r, 0)

                pltpu.async_copy(xs.at[idxg.at[0]], msg.at[0], g0)

                def t_body(t, carry):
                    j0 = 2 * t
                    j1 = j0 + 1
                    pltpu.make_async_copy(
                        xs.at[idxg.at[j0]], msg.at[0], g0).wait()
                    pltpu.async_copy(xs.at[idxg.at[j1]], msg.at[1], g1)
                    pltpu.sync_copy(msg.at[0], acc.at[idxd.at[j0]], add=True)
                    pltpu.make_async_copy(
                        xs.at[idxg.at[j1]], msg.at[1], g1).wait()

                    @pl.when(t < W // 2 - 1)
                    def _next():
                        pltpu.async_copy(xs.at[idxg.at[j0 + 2]], msg.at[0], g0)

                    pltpu.sync_copy(msg.at[1], acc.at[idxd.at[j1]], add=True)
                    return carry

                lax.fori_loop(0, W // 2, t_body, 0)

            plsc.subcore_barrier()

            def out_b(i, carry):
                pltpu.sync_copy(acc.at[pl.ds(s * STRIPE + i * B, B)], msg.at[0])
                pltpu.sync_copy(msg.at[0],
                                out.at[pl.ds(base + s * STRIPE + i * B, B)])
                return carry

            lax.fori_loop(0, STRIPE // B, out_b, 0)

    return _scatter


_scatter2 = _make_scatter(2)
_scatter4 = _make_scatter(4)


def _dis_of(degp_ref):
    deg = jnp.sum(degp_ref[0] + degp_ref[1], axis=1) * (1.0 / DEGW) + 1.0
    return 1.0 / jnp.sqrt(deg)


def _k0_body(degp_ref, x_ref, xs1_ref):
    dis = _dis_of(degp_ref)
    xs = x_ref[...] * dis[:, None]
    for k in range(2):
        xs1_ref[k] = xs[:, k * CW:(k + 1) * CW]


def _ka_body(degp_ref, acc1_ref, w1_ref, b1_ref, xs2_ref):
    dis = _dis_of(degp_ref)
    s1 = jnp.concatenate([acc1_ref[k] for k in range(2)], axis=1) * dis[:, None]
    h = jnp.dot(s1, w1_ref[...], preferred_element_type=jnp.float32) + b1_ref[...]
    h = jnp.maximum(h, 0.0)
    xs2 = h * dis[:, None]
    for k in range(4):
        xs2_ref[k] = xs2[:, k * CW:(k + 1) * CW]


def _kb_body(degp_ref, acc2_ref, w2_ref, b2_ref, wr_ref, bo_ref, out_ref):
    dis = _dis_of(degp_ref)
    s2 = jnp.concatenate([acc2_ref[k] for k in range(4)], axis=1) * dis[:, None]
    h = jnp.dot(s2, w2_ref[...], preferred_element_type=jnp.float32) + b2_ref[...]
    h = jnp.maximum(h, 0.0)
    hg = h.reshape(BLK // 100, 100, 512)
    vals = jnp.sum(hg * wr_ref[...][None], axis=(1, 2))
    out_ref[0] = jnp.broadcast_to(vals[:, None], (BLK // 100, 128)) + bo_ref[0, 0]


_degp_spec = pl.BlockSpec((2, BLK, DEGW), lambda i: (0, i, 0))


def _k0(degp, x):
    return pl.pallas_call(
        _k0_body,
        grid=(N // BLK,),
        in_specs=[_degp_spec, pl.BlockSpec((BLK, 256), lambda i: (i, 0))],
        out_specs=pl.BlockSpec((2, BLK, CW), lambda i: (0, i, 0)),
        out_shape=jax.ShapeDtypeStruct((2, NPAD, CW), jnp.float32),
    )(degp, x)


def _ka(degp, acc1, W1, b1):
    return pl.pallas_call(
        _ka_body,
        grid=(N // BLK,),
        in_specs=[
            _degp_spec,
            pl.BlockSpec((2, BLK, CW), lambda i: (0, i, 0)),
            pl.BlockSpec((256, 512), lambda i: (0, 0)),
            pl.BlockSpec((1, 512), lambda i: (0, 0)),
        ],
        out_specs=pl.BlockSpec((4, BLK, CW), lambda i: (0, i, 0)),
        out_shape=jax.ShapeDtypeStruct((4, NPAD, CW), jnp.float32),
    )(degp, acc1, W1, b1)


def _kb(degp, acc2, W2, b2, wr, bo):
    return pl.pallas_call(
        _kb_body,
        grid=(N // BLK,),
        in_specs=[
            _degp_spec,
            pl.BlockSpec((4, BLK, CW), lambda i: (0, i, 0)),
            pl.BlockSpec((512, 512), lambda i: (0, 0)),
            pl.BlockSpec((1, 512), lambda i: (0, 0)),
            pl.BlockSpec((100, 512), lambda i: (0, 0)),
            pl.BlockSpec((1, 1), lambda i: (0, 0)),
        ],
        out_specs=pl.BlockSpec((1, BLK // 100, 128), lambda i: (i, 0, 0)),
        out_shape=jax.ShapeDtypeStruct((N // BLK, BLK // 100, 128), jnp.float32),
    )(degp, acc2, W2, b2, wr, bo)


def kernel(x, edge_index, W1, b1, W2, b2, W_out, b_out):
    src = edge_index[0].astype(jnp.int32)
    dst = edge_index[1].astype(jnp.int32)
    npad = EPAD - src.shape[0]
    src_p = jnp.concatenate([src, jnp.zeros((npad,), jnp.int32)])
    dst_p = jnp.concatenate([dst, jnp.full((npad,), N, jnp.int32)])
    srcr = src_p.reshape(16, NB, B)
    dstr = dst_p.reshape(16, NB, B)
    ddeg = dst_p.reshape(32, NB // 2, B)

    ones_h = jnp.ones((B, DEGW), jnp.float32)
    zero_h = jnp.zeros((B, DEGW), jnp.float32)
    degp = _deg_kernel(ddeg, ones_h, zero_h).reshape(2, NPAD, DEGW)
    xs1 = _k0(degp, x).reshape(2 * NPAD, CW)
    acc1 = _scatter2(xs1, srcr, dstr).reshape(2, NPAD, CW)
    xs2 = _ka(degp, acc1, W1, b1.reshape(1, 512)).reshape(4 * NPAD, CW)
    acc2 = _scatter4(xs2, srcr, dstr).reshape(4, NPAD, CW)
    res = _kb(degp, acc2, W2, b2.reshape(1, 512),
              W_out.reshape(100, 512), b_out.reshape(1, 1))
    return res.reshape(100, 128)[:, :1]
